# Initial kernel scaffold; baseline (speedup 1.0000x reference)
import jax
import jax.numpy as jnp
from jax.experimental import pallas as pl


def kernel(color_point_fea, color_point_link, color_features_expand, fc11, fc21, fc31, fc41, fc22, fc23, fc_end):
    raise NotImplementedError("write your pallas kernel here")


# speedup vs baseline: 10.0967x; 10.0967x over previous
import functools

import jax
import jax.numpy as jnp
from jax import lax
from jax.experimental import pallas as pl
from jax.experimental.pallas import tpu as pltpu
from jax.experimental.pallas import tpu_sc as plsc

_LANES = 16


def _collapse(p):
    W1, b1, W2, b2, W3, b3 = p
    A = (W3 @ W2 @ W1).T
    b = W3 @ (W2 @ b1 + b2) + b3
    return A.astype(jnp.float32), b.reshape(1, -1).astype(jnp.float32)


def _pad_rows(A, rows):
    return jnp.pad(A, ((0, rows - A.shape[0]), (0, 0)))


def _sc_gather(table, idx, e_pad, n_chunks):
    nw = 32
    b_w = e_pad // nw
    chunk = b_w // n_chunks
    mesh = plsc.VectorSubcoreMesh(
        core_axis_name="c", subcore_axis_name="s", num_cores=2, num_subcores=16
    )

    def body(table_hbm, idx_hbm, out_hbm, idx_v, rows_v, sem):
        wid = lax.axis_index("s") * 2 + lax.axis_index("c")
        base = wid * b_w
        for c in range(n_chunks):
            off = base + c * chunk
            pltpu.sync_copy(idx_hbm.at[pl.ds(off, chunk)], idx_v)
            pltpu.async_copy(table_hbm.at[idx_v], rows_v, sem).wait()
            pltpu.sync_copy(rows_v, out_hbm.at[pl.ds(off, chunk)])

    kfn = pl.kernel(
        body,
        out_type=jax.ShapeDtypeStruct((e_pad, _LANES), jnp.float32),
        mesh=mesh,
        scratch_types=[
            pltpu.VMEM((chunk,), jnp.int32),
            pltpu.VMEM((chunk, _LANES), jnp.float32),
            pltpu.SemaphoreType.DMA,
        ],
        compiler_params=pltpu.CompilerParams(use_tc_tiling_on_sc=False),
    )
    return kfn(table, idx)


def _dense_body(*refs):
    (x_ref, g0, g1, g2, g3, g4, g5, g6, g7, g8, cfe_ref,
     a11, b11, a21, b21, a31, b31, a41, b41, a22, c22, a23, b23,
     e4, e3, e2, e1, e0, aend, bend, out_ref) = refs
    grefs = (g0, g1, g2, g3, g4, g5, g6, g7, g8)

    def dot(a, b):
        return jnp.dot(a, b, preferred_element_type=jnp.float32)

    relu = lambda t: jnp.maximum(t, 0.0)

    x = x_ref[...]
    f1 = relu(dot(x, a11[...]) + b11[...])
    f2 = relu(dot(f1, a21[...]) + b21[...])
    f3 = relu(dot(f2, a31[...]) + b31[...])
    f4 = relu(dot(f3, a41[...]) + b41[...])
    v = dot(x, a22[...]) - c22[...]

    a23_full = a23[...]
    acc = jnp.zeros((x.shape[0], a23_full.shape[1]), jnp.float32)
    for j in range(9):
        gj = grefs[j][...]
        gate = relu(dot(gj, a22[...]) - v)
        f1g = relu(dot(gj, a11[...]) + b11[...])
        f2g = relu(dot(f1g, a21[...]) + b21[...])
        acc = acc + dot(f2g * gate, a23_full[j * 54:(j + 1) * 54, :])
    f21 = relu(acc + b23[...])

    out1 = relu(
        dot(f4, e4[...]) + dot(f3, e3[...]) + dot(f21, e2[...])
        + dot(f1, e1[...]) + dot(x, e0[...]) + bend[...]
    )
    cfeo = relu(dot(cfe_ref[...], aend[...]) + bend[...])
    out_ref[...] = jnp.concatenate([out1, cfeo], axis=1)


def kernel(color_point_fea, color_point_link, color_features_expand,
           fc11, fc21, fc31, fc41, fc22, fc23, fc_end):
    n, m = color_point_fea.shape
    assert m == 9

    A11, b11 = _collapse(fc11)
    A21, b21 = _collapse(fc21)
    A31, b31 = _collapse(fc31)
    A41, b41 = _collapse(fc41)
    A22, c22b = _collapse(fc22)
    A23, b23 = _collapse(fc23)
    Aend, bend = _collapse(fc_end)
    A11p = _pad_rows(A11, _LANES)
    A22p = _pad_rows(A22, _LANES)
    negc22 = -c22b
    E4 = Aend[0:6]
    E3 = Aend[6:24]
    E2 = Aend[24:78]
    E1 = Aend[78:96]
    E0 = _pad_rows(Aend[96:102], _LANES)

    x_pad = jnp.pad(color_point_fea, ((0, 0), (0, _LANES - m)))
    e = n * m
    e_pad = -(-e // 256) * 256
    link_t = jnp.transpose(color_point_link.reshape(n, m)).reshape(-1)
    link_t = link_t.astype(jnp.int32)
    link_t = jnp.concatenate(
        [link_t, jnp.arange(e_pad - e, dtype=jnp.int32)])

    b_w = e_pad // 32
    n_chunks = next(k for k in range(1, 65)
                    if b_w % k == 0 and (b_w // k) % 8 == 0
                    and (b_w // k) * (4 * _LANES + 4) <= 450_000)
    G = _sc_gather(x_pad, link_t, e_pad, n_chunks)

    R = 2000
    assert n % R == 0
    nb = n // R
    bspec = lambda shape, imap: pl.BlockSpec(shape, imap)
    wspec = lambda w: pl.BlockSpec(w.shape, lambda i: (0, 0))
    g_specs = [
        pl.BlockSpec((R, _LANES), functools.partial(lambda j, i: (j * nb + i, 0), j))
        for j in range(m)
    ]
    weights = (A11p, b11, A21, b21, A31, b31, A41, b41, A22p, negc22,
               A23, b23, E4, E3, E2, E1, E0, Aend, bend)
    out = pl.pallas_call(
        _dense_body,
        grid=(nb,),
        in_specs=[bspec((R, _LANES), lambda i: (i, 0))]
        + g_specs
        + [bspec((R, 102), lambda i: (i, 0))]
        + [wspec(w) for w in weights],
        out_specs=bspec((R, 102), lambda i: (i, 0)),
        out_shape=jax.ShapeDtypeStruct((n, 102), jnp.float32),
    )(x_pad, *([G] * m), color_features_expand, *weights)
    return out


# speedup vs baseline: 17.9311x; 1.7759x over previous
import jax
import jax.numpy as jnp
from jax import lax
from jax.experimental import pallas as pl
from jax.experimental.pallas import tpu as pltpu
from jax.experimental.pallas import tpu_sc as plsc

_L = 16
_M = 9


def _collapse(p):
    W1, b1, W2, b2, W3, b3 = p
    A = (W3 @ W2 @ W1).T
    b = W3 @ (W2 @ b1 + b2) + b3
    return A.astype(jnp.float32), b.reshape(1, -1).astype(jnp.float32)


def _pad_rows(A, rows):
    return jnp.pad(A, ((0, rows - A.shape[0]), (0, 0)))


def _blockdiag9(A):
    k, f = A.shape
    out = jnp.zeros((_M * k, _M * f), jnp.float32)
    for j in range(_M):
        out = out.at[j * k:(j + 1) * k, j * f:(j + 1) * f].set(A)
    return out


def _tile9(b):
    return jnp.tile(b, (1, _M))


def _sc_gather(table, idx, n_pad, c_nodes, n_chunks):
    nodes_w = n_pad // 32
    mesh = plsc.VectorSubcoreMesh(
        core_axis_name="c", subcore_axis_name="s", num_cores=2, num_subcores=16
    )

    def body(table_hbm, idx_hbm, out_hbm, idx_v, rows_v, sem):
        wid = lax.axis_index("s") * 2 + lax.axis_index("c")
        for c in range(n_chunks):
            e0 = (wid * nodes_w + c * c_nodes) * _M
            pltpu.sync_copy(idx_hbm.at[pl.ds(e0, c_nodes * _M)], idx_v)
            pltpu.async_copy(table_hbm.at[idx_v], rows_v, sem).wait()
            pltpu.sync_copy(rows_v, out_hbm.at[pl.ds(e0, c_nodes * _M)])

    kfn = pl.kernel(
        body,
        out_type=jax.ShapeDtypeStruct((n_pad * _M, _L), jnp.float32),
        mesh=mesh,
        scratch_types=[
            pltpu.VMEM((c_nodes * _M,), jnp.int32),
            pltpu.VMEM((c_nodes * _M, _L), jnp.float32),
            pltpu.SemaphoreType.DMA,
        ],
        compiler_params=pltpu.CompilerParams(use_tc_tiling_on_sc=False),
    )
    return kfn(table, idx)


def _dense_body(*refs):
    (x_ref, g_ref, cfe_ref,
     a11, b11, a21, b21, a31, b31, a41, b41,
     w22bd, a22n9, c22t, w11bd, b11t, w21bd, b21t, a23, b23,
     ccw, aend, bend, out_ref) = refs

    def dot(a, b):
        return jnp.dot(a, b, preferred_element_type=jnp.float32)

    relu = lambda t: jnp.maximum(t, 0.0)

    x = x_ref[...]
    g = g_ref[...]
    f1 = relu(dot(x, a11[...]) + b11[...])
    f2 = relu(dot(f1, a21[...]) + b21[...])
    f3 = relu(dot(f2, a31[...]) + b31[...])
    f4 = relu(dot(f3, a41[...]) + b41[...])

    gates = relu(dot(g, w22bd[...]) + dot(x, a22n9[...]) + c22t[...])
    f1g = relu(dot(g, w11bd[...]) + b11t[...])
    f2g = relu(dot(f1g, w21bd[...]) + b21t[...])
    f21 = relu(dot(f2g * gates, a23[...]) + b23[...])

    cc = jnp.concatenate([f4, f3, f21, f1, x], axis=1)
    out1 = relu(dot(cc, ccw[...]) + bend[...])
    cfeo = relu(dot(cfe_ref[...], aend[...]) + bend[...])
    out_ref[...] = jnp.concatenate([out1, cfeo], axis=1)


def kernel(color_point_fea, color_point_link, color_features_expand,
           fc11, fc21, fc31, fc41, fc22, fc23, fc_end):
    n, m = color_point_fea.shape
    assert m == _M

    A11, b11 = _collapse(fc11)
    A21, b21 = _collapse(fc21)
    A31, b31 = _collapse(fc31)
    A41, b41 = _collapse(fc41)
    A22, c22 = _collapse(fc22)
    A23, b23 = _collapse(fc23)
    Aend, bend = _collapse(fc_end)
    A11p = _pad_rows(A11, _L)
    A22p = _pad_rows(A22, _L)
    W22bd = _blockdiag9(A22p)
    A22n9 = jnp.tile(-A22p, (1, _M))
    C22t = _tile9(c22)
    W11bd = _blockdiag9(A11p)
    B11t = _tile9(b11)
    W21bd = _blockdiag9(A21)
    B21t = _tile9(b21)
    CCW = jnp.concatenate(
        [Aend[0:6], Aend[6:24], Aend[24:78], Aend[78:96],
         _pad_rows(Aend[96:102], _L)], axis=0)

    x_pad = jnp.pad(color_point_fea, ((0, 0), (0, _L - m)))
    n_pad = -(-n // 256) * 256
    e, e_pad = n * m, n_pad * m
    idx = jnp.concatenate([
        color_point_link.astype(jnp.int32),
        jnp.arange(e_pad - e, dtype=jnp.int32) % n,
    ])

    nodes_w = n_pad // 32
    c_nodes = next(k for k in range(nodes_w, 0, -1)
                   if nodes_w % k == 0 and k % 8 == 0
                   and k * _M * (4 * _L + 4) <= 450_000)
    G = _sc_gather(x_pad, idx, n_pad, c_nodes, nodes_w // c_nodes)
    G = G.reshape(n_pad, _M * _L)

    R = 2000
    assert n % R == 0
    weights = (A11p, b11, A21, b21, A31, b31, A41, b41,
               W22bd, A22n9, C22t, W11bd, B11t, W21bd, B21t, A23, b23,
               CCW, Aend, bend)
    bspec = lambda shape: pl.BlockSpec(shape, lambda i: (i, 0))
    wspec = lambda w: pl.BlockSpec(w.shape, lambda i: (0, 0))
    out = pl.pallas_call(
        _dense_body,
        grid=(n // R,),
        in_specs=[bspec((R, _L)), bspec((R, _M * _L)), bspec((R, 102))]
        + [wspec(w) for w in weights],
        out_specs=bspec((R, 102)),
        out_shape=jax.ShapeDtypeStruct((n, 102), jnp.float32),
    )(x_pad, G, color_features_expand, *weights)
    return out


# speedup vs baseline: 18.4217x; 1.0274x over previous
import jax
import jax.numpy as jnp
from jax import lax
from jax.experimental import pallas as pl
from jax.experimental.pallas import tpu as pltpu
from jax.experimental.pallas import tpu_sc as plsc

_L = 16
_M = 9


def _collapse(p):
    W1, b1, W2, b2, W3, b3 = p
    A = (W3 @ W2 @ W1).T
    b = W3 @ (W2 @ b1 + b2) + b3
    return A.astype(jnp.float32), b.reshape(1, -1).astype(jnp.float32)


def _pad_rows(A, rows):
    return jnp.pad(A, ((0, rows - A.shape[0]), (0, 0)))


def _blockdiag9(A):
    k, f = A.shape
    out = jnp.zeros((_M * k, _M * f), jnp.float32)
    for j in range(_M):
        out = out.at[j * k:(j + 1) * k, j * f:(j + 1) * f].set(A)
    return out


def _tile9(b):
    return jnp.tile(b, (1, _M))


def _sc_gather(table, idx, n_pad, c_nodes, n_chunks):
    nodes_w = n_pad // 32
    mesh = plsc.VectorSubcoreMesh(
        core_axis_name="c", subcore_axis_name="s", num_cores=2, num_subcores=16
    )

    def body(table_hbm, idx_hbm, out_hbm, idx_v, rows_v, sem):
        wid = lax.axis_index("s") * 2 + lax.axis_index("c")
        for c in range(n_chunks):
            e0 = (wid * nodes_w + c * c_nodes) * _M
            pltpu.sync_copy(idx_hbm.at[pl.ds(e0, c_nodes * _M)], idx_v)
            pltpu.async_copy(table_hbm.at[idx_v], rows_v, sem).wait()
            pltpu.sync_copy(rows_v, out_hbm.at[pl.ds(e0, c_nodes * _M)])

    kfn = pl.kernel(
        body,
        out_type=jax.ShapeDtypeStruct((n_pad * _M, _L), jnp.float32),
        mesh=mesh,
        scratch_types=[
            pltpu.VMEM((c_nodes * _M,), jnp.int32),
            pltpu.VMEM((c_nodes * _M, _L), jnp.float32),
            pltpu.SemaphoreType.DMA,
        ],
        compiler_params=pltpu.CompilerParams(use_tc_tiling_on_sc=False),
    )
    return kfn(table, idx)


def _dense_body(*refs):
    (x_ref, g_ref, cfe_ref,
     a11, b11, a21, b21, a31, b31, a41, b41,
     w22bd, a22n9, c22t, w11bd, b11t, w21bd, b21t, a23, b23,
     ccw, aend, bend, out_ref) = refs

    def dot(a, b):
        return jnp.dot(a, b, preferred_element_type=jnp.float32)

    relu = lambda t: jnp.maximum(t, 0.0)

    x = x_ref[...]
    g = g_ref[...]
    f1 = relu(dot(x, a11[...]) + b11[...])
    f2 = relu(dot(f1, a21[...]) + b21[...])
    f3 = relu(dot(f2, a31[...]) + b31[...])
    f4 = relu(dot(f3, a41[...]) + b41[...])

    gates = relu(dot(g, w22bd[...]) + dot(x, a22n9[...]) + c22t[...])
    f1g = relu(dot(g, w11bd[...]) + b11t[...])
    f2g = relu(dot(f1g, w21bd[...]) + b21t[...])
    f21 = relu(dot(f2g * gates, a23[...]) + b23[...])

    cc = jnp.concatenate([f4, f3, f21, f1, x], axis=1)
    out1 = relu(dot(cc, ccw[...]) + bend[...])
    cfeo = relu(dot(cfe_ref[...], aend[...]) + bend[...])
    out_ref[...] = jnp.concatenate([out1, cfeo], axis=1)


def kernel(color_point_fea, color_point_link, color_features_expand,
           fc11, fc21, fc31, fc41, fc22, fc23, fc_end):
    n, m = color_point_fea.shape
    assert m == _M

    A11, b11 = _collapse(fc11)
    A21, b21 = _collapse(fc21)
    A31, b31 = _collapse(fc31)
    A41, b41 = _collapse(fc41)
    A22, c22 = _collapse(fc22)
    A23, b23 = _collapse(fc23)
    Aend, bend = _collapse(fc_end)
    A11p = _pad_rows(A11, _L)
    A22p = _pad_rows(A22, _L)
    W22bd = _blockdiag9(A22p)
    A22n9 = jnp.tile(-A22p, (1, _M))
    C22t = _tile9(c22)
    W11bd = _blockdiag9(A11p)
    B11t = _tile9(b11)
    W21bd = _blockdiag9(A21)
    B21t = _tile9(b21)
    CCW = jnp.concatenate(
        [Aend[0:6], Aend[6:24], Aend[24:78], Aend[78:96],
         _pad_rows(Aend[96:102], _L)], axis=0)

    x_pad = jnp.pad(color_point_fea, ((0, 0), (0, _L - m)))
    n_pad = -(-n // 256) * 256
    e, e_pad = n * m, n_pad * m
    idx = jnp.concatenate([
        color_point_link.astype(jnp.int32),
        jnp.arange(e_pad - e, dtype=jnp.int32) % n,
    ])

    nodes_w = n_pad // 32
    c_nodes = next(k for k in range(nodes_w, 0, -1)
                   if nodes_w % k == 0 and k % 8 == 0
                   and k * _M * (4 * _L + 4) <= 450_000)
    G = _sc_gather(x_pad, idx, n_pad, c_nodes, nodes_w // c_nodes)
    G = G.reshape(n_pad, _M * _L)

    R = 4000
    assert n % R == 0
    weights = (A11p, b11, A21, b21, A31, b31, A41, b41,
               W22bd, A22n9, C22t, W11bd, B11t, W21bd, B21t, A23, b23,
               CCW, Aend, bend)
    bspec = lambda shape: pl.BlockSpec(shape, lambda i: (i, 0))
    wspec = lambda w: pl.BlockSpec(w.shape, lambda i: (0, 0))
    out = pl.pallas_call(
        _dense_body,
        grid=(n // R,),
        in_specs=[bspec((R, _L)), bspec((R, _M * _L)), bspec((R, 102))]
        + [wspec(w) for w in weights],
        out_specs=bspec((R, 102)),
        out_shape=jax.ShapeDtypeStruct((n, 102), jnp.float32),
    )(x_pad, G, color_features_expand, *weights)
    return out
